# Initial kernel scaffold; baseline (speedup 1.0000x reference)
import jax
import jax.numpy as jnp
from jax.experimental import pallas as pl


def kernel(x, edge_index, batch, W1, b1, g1, be1, W2, b2, g2, be2, W3, b3, g3, be3, W4, b4, P1w, P1b, P2w, P2b, P3w, P3b):
    raise NotImplementedError("write your pallas kernel here")


# speedup vs baseline: 9.2036x; 9.2036x over previous
import functools

import jax
import jax.numpy as jnp
from jax import lax
from jax.experimental import pallas as pl
from jax.experimental.pallas import tpu as pltpu
from jax.experimental.pallas import tpu_sc as plsc

NN = 100000
EE = 1600000
NG = 64
NSUB = 16
NCORE = 2
EPB = 2000
E_PER_SUB = EE // NSUB
NBLK = E_PER_SUB // EPB
BUF = 2304
BATCH = 128

_PASS_SIZES = {
    16: (50000,),
    32: (50000,),
    64: (25600, 24400),
    128: (12800, 12800, 12800, 11600),
}


def _make_prop(D, ones_mode):
    pass_sizes = _PASS_SIZES[D]
    r_alloc = max(pass_sizes) + 16
    dummy = r_alloc - 1
    mesh = plsc.VectorSubcoreMesh(core_axis_name="c", subcore_axis_name="s")

    @functools.partial(
        pl.kernel,
        out_type=jax.ShapeDtypeStruct((NN, D), jnp.float32),
        mesh=mesh,
        compiler_params=pltpu.CompilerParams(use_tc_tiling_on_sc=False, needs_layout_passes=False),
        scratch_types=[
            pltpu.VMEM((EPB,), jnp.int32),
            pltpu.VMEM((EPB,), jnp.int32),
            pltpu.VMEM((BUF,), jnp.int32),
            pltpu.VMEM((BUF,), jnp.int32),
            pltpu.VMEM((BATCH,), jnp.int32),
            pltpu.VMEM((BATCH,), jnp.int32),
            pltpu.VMEM((BATCH, D), jnp.float32),
            pltpu.VMEM_SHARED((r_alloc, D), jnp.float32),
            pltpu.SemaphoreType.DMA,
        ],
    )
    def prop(hp, src_e, dst_e, out, dstv, srcv, srcbuf, dstbuf, idxs, idxd,
             rowbuf, acc, sem):
        c = lax.axis_index("c")
        s = lax.axis_index("s")
        core_lo = c * (NN // NCORE)
        ebase = s * E_PER_SUB

        if ones_mode:
            one16 = jnp.ones((16,), jnp.float32)
            for q in range(BATCH):
                rowbuf[q, :] = one16

        def fire(j, _):
            base = j * BATCH
            for q in range(BATCH // 16):
                idxs[pl.ds(q * 16, 16)] = srcbuf[pl.ds(base + q * 16, 16)]
                idxd[pl.ds(q * 16, 16)] = dstbuf[pl.ds(base + q * 16, 16)]
            if not ones_mode:
                pltpu.async_copy(hp.at[idxs], rowbuf, sem).wait()
            pltpu.sync_copy(rowbuf, acc.at[idxd], add=True)
            return 0

        off = 0
        for psize in pass_sizes:
            lo = core_lo + off
            hi = lo + psize
            rps = -(-(psize // NSUB) // 8) * 8
            rlast = psize - (NSUB - 1) * rps

            @pl.when(s < NSUB - 1)
            def _():
                pltpu.sync_copy(hp.at[pl.ds(lo + s * rps, rps)],
                                acc.at[pl.ds(s * rps, rps)])

            @pl.when(s == NSUB - 1)
            def _():
                pltpu.sync_copy(hp.at[pl.ds(lo + (NSUB - 1) * rps, rlast)],
                                acc.at[pl.ds((NSUB - 1) * rps, rlast)])

            plsc.subcore_barrier()

            def blk_body(b, fill):
                e0 = ebase + b * EPB
                pltpu.sync_copy(dst_e.at[pl.ds(e0, EPB)], dstv)
                pltpu.sync_copy(src_e.at[pl.ds(e0, EPB)], srcv)

                def scan_body(i, f):
                    dd = dstv[pl.ds(i * 16, 16)]
                    ss = srcv[pl.ds(i * 16, 16)]
                    m = (dd >= lo) & (dd < hi)
                    mi = jnp.where(m, 1, 0)
                    pos = f + plsc.cumsum(mi) - 1
                    plsc.store_scatter(srcbuf, [pos], ss, mask=m)
                    plsc.store_scatter(dstbuf, [pos], dd - lo, mask=m)
                    return f + jnp.sum(mi)

                fill = lax.fori_loop(0, EPB // 16, scan_body, fill)
                nb = fill // BATCH
                lax.fori_loop(0, nb, fire, 0)
                rem = nb * BATCH
                for q in range(BATCH // 16):
                    sv = srcbuf[pl.ds(rem + q * 16, 16)]
                    dv = dstbuf[pl.ds(rem + q * 16, 16)]
                    srcbuf[pl.ds(q * 16, 16)] = sv
                    dstbuf[pl.ds(q * 16, 16)] = dv
                return fill - rem

            fill = lax.fori_loop(0, NBLK, blk_body, 0)

            zero16 = jnp.zeros((16,), jnp.int32)
            dum16 = jnp.full((16,), dummy, jnp.int32)
            for q in range(BATCH // 16):
                srcbuf[pl.ds(fill + q * 16, 16)] = zero16
                dstbuf[pl.ds(fill + q * 16, 16)] = dum16
            lax.fori_loop(0, (fill + BATCH - 1) // BATCH, fire, 0)

            plsc.subcore_barrier()

            @pl.when(s < NSUB - 1)
            def _():
                pltpu.sync_copy(acc.at[pl.ds(s * rps, rps)],
                                out.at[pl.ds(lo + s * rps, rps)])

            @pl.when(s == NSUB - 1)
            def _():
                pltpu.sync_copy(acc.at[pl.ds((NSUB - 1) * rps, rlast)],
                                out.at[pl.ds(lo + (NSUB - 1) * rps, rlast)])

            plsc.subcore_barrier()
            off += psize

    return prop


_prop_deg = _make_prop(16, True)
_prop128 = _make_prop(128, False)
_prop64 = _make_prop(64, False)
_prop32 = _make_prop(32, False)


_BLK = 1000
_GRID = NN // _BLK


def _mm_body(x_ref, w_ref, deg_ref, o_ref):
    dis = lax.rsqrt(deg_ref[...])
    o_ref[...] = dis * jnp.dot(x_ref[...], w_ref[...],
                               preferred_element_type=jnp.float32)


def _matmul_scale(x, W, degcol):
    din, dout = W.shape
    return pl.pallas_call(
        _mm_body,
        grid=(_GRID,),
        in_specs=[
            pl.BlockSpec((_BLK, din), lambda i: (i, 0)),
            pl.BlockSpec((din, dout), lambda i: (0, 0)),
            pl.BlockSpec((_BLK, 1), lambda i: (i, 0)),
        ],
        out_specs=pl.BlockSpec((_BLK, dout), lambda i: (i, 0)),
        out_shape=jax.ShapeDtypeStruct((NN, dout), jnp.float32),
    )(x, W, degcol)


def _red_body(p_ref, deg_ref, b_ref, o_ref):
    @pl.when(pl.program_id(0) == 0)
    def _():
        o_ref[...] = jnp.zeros_like(o_ref)

    dis = lax.rsqrt(deg_ref[...])
    y = dis * p_ref[...] + b_ref[...]
    s1 = jnp.sum(y, axis=0, keepdims=True)
    s2 = jnp.sum(y * y, axis=0, keepdims=True)
    o_ref[...] += jnp.concatenate(
        [s1, s2, jnp.zeros((6, y.shape[1]), jnp.float32)], axis=0)


def _reduce_sums(P, degcol, brow):
    d = P.shape[1]
    return pl.pallas_call(
        _red_body,
        grid=(_GRID,),
        in_specs=[
            pl.BlockSpec((_BLK, d), lambda i: (i, 0)),
            pl.BlockSpec((_BLK, 1), lambda i: (i, 0)),
            pl.BlockSpec((1, d), lambda i: (0, 0)),
        ],
        out_specs=pl.BlockSpec((8, d), lambda i: (0, 0)),
        out_shape=jax.ShapeDtypeStruct((8, d), jnp.float32),
    )(P, degcol, brow)


def _norm_body(p_ref, deg_ref, b_ref, g_ref, be_ref, s_ref, o_ref):
    dis = lax.rsqrt(deg_ref[...])
    y = dis * p_ref[...] + b_ref[...]
    m = s_ref[0:1, :] * (1.0 / NN)
    v = s_ref[1:2, :] * (1.0 / NN) - m * m
    yn = (y - m) * lax.rsqrt(v + 1e-5) * g_ref[...] + be_ref[...]
    o_ref[...] = jnp.maximum(yn, 0.0)


def _norm_res_body(p_ref, deg_ref, b_ref, g_ref, be_ref, s_ref, r_ref, o_ref):
    dis = lax.rsqrt(deg_ref[...])
    y = dis * p_ref[...] + b_ref[...]
    m = s_ref[0:1, :] * (1.0 / NN)
    v = s_ref[1:2, :] * (1.0 / NN) - m * m
    yn = (y - m) * lax.rsqrt(v + 1e-5) * g_ref[...] + be_ref[...]
    o_ref[...] = jnp.maximum(yn, 0.0) + r_ref[...]


def _bn_relu(P, degcol, brow, grow, berow, sums, res=None):
    d = P.shape[1]
    specs = [
        pl.BlockSpec((_BLK, d), lambda i: (i, 0)),
        pl.BlockSpec((_BLK, 1), lambda i: (i, 0)),
        pl.BlockSpec((1, d), lambda i: (0, 0)),
        pl.BlockSpec((1, d), lambda i: (0, 0)),
        pl.BlockSpec((1, d), lambda i: (0, 0)),
        pl.BlockSpec((8, d), lambda i: (0, 0)),
    ]
    args = [P, degcol, brow, grow, berow, sums]
    body = _norm_body
    if res is not None:
        specs.append(pl.BlockSpec((_BLK, d), lambda i: (i, 0)))
        args.append(res)
        body = _norm_res_body
    return pl.pallas_call(
        body,
        grid=(_GRID,),
        in_specs=specs,
        out_specs=pl.BlockSpec((_BLK, d), lambda i: (i, 0)),
        out_shape=jax.ShapeDtypeStruct((NN, d), jnp.float32),
    )(*args)


def _relu_body(p_ref, deg_ref, b_ref, o_ref):
    dis = lax.rsqrt(deg_ref[...])
    o_ref[...] = jnp.maximum(dis * p_ref[...] + b_ref[...], 0.0)


def _scale_relu(P, degcol, brow):
    d = P.shape[1]
    return pl.pallas_call(
        _relu_body,
        grid=(_GRID,),
        in_specs=[
            pl.BlockSpec((_BLK, d), lambda i: (i, 0)),
            pl.BlockSpec((_BLK, 1), lambda i: (i, 0)),
            pl.BlockSpec((1, d), lambda i: (0, 0)),
        ],
        out_specs=pl.BlockSpec((_BLK, d), lambda i: (i, 0)),
        out_shape=jax.ShapeDtypeStruct((NN, d), jnp.float32),
    )(P, degcol, brow)


def _pool_body(x4_ref, x3_ref, bat_ref, s4_ref, s3_ref, cnt_ref):
    @pl.when(pl.program_id(0) == 0)
    def _():
        s4_ref[...] = jnp.zeros_like(s4_ref)
        s3_ref[...] = jnp.zeros_like(s3_ref)
        cnt_ref[...] = jnp.zeros_like(cnt_ref)

    oh = (bat_ref[...] == lax.broadcasted_iota(jnp.int32, (1, NG), 1)
          ).astype(jnp.float32)
    dnum = (((0,), (0,)), ((), ()))
    s4_ref[...] += lax.dot_general(oh, x4_ref[...], dnum,
                                   preferred_element_type=jnp.float32)
    s3_ref[...] += lax.dot_general(oh, x3_ref[...], dnum,
                                   preferred_element_type=jnp.float32)
    cnt_ref[...] += lax.dot_general(
        oh, jnp.ones((oh.shape[0], 8), jnp.float32), dnum,
        preferred_element_type=jnp.float32)


def _pool(x4, x3, batch2d):
    return pl.pallas_call(
        _pool_body,
        grid=(_GRID,),
        in_specs=[
            pl.BlockSpec((_BLK, 32), lambda i: (i, 0)),
            pl.BlockSpec((_BLK, 64), lambda i: (i, 0)),
            pl.BlockSpec((_BLK, 1), lambda i: (i, 0)),
        ],
        out_specs=[
            pl.BlockSpec((NG, 32), lambda i: (0, 0)),
            pl.BlockSpec((NG, 64), lambda i: (0, 0)),
            pl.BlockSpec((NG, 8), lambda i: (0, 0)),
        ],
        out_shape=[
            jax.ShapeDtypeStruct((NG, 32), jnp.float32),
            jax.ShapeDtypeStruct((NG, 64), jnp.float32),
            jax.ShapeDtypeStruct((NG, 8), jnp.float32),
        ],
    )(x4, x3, batch2d)


def _head_body(s4_ref, s3_ref, cnt_ref, w1_ref, b1_ref, w2_ref, b2_ref,
               w3_ref, b3_ref, o_ref):
    cnt = cnt_ref[...][:, 0:1]
    emb = jnp.concatenate([s4_ref[...], s3_ref[...]], axis=1) / cnt
    h = jnp.maximum(jnp.dot(emb, w1_ref[...],
                            preferred_element_type=jnp.float32)
                    + b1_ref[...], 0.0)
    h = jnp.maximum(jnp.dot(h, w2_ref[...],
                            preferred_element_type=jnp.float32)
                    + b2_ref[...], 0.0)
    o_ref[...] = jnp.dot(h, w3_ref[...],
                         preferred_element_type=jnp.float32) + b3_ref[...]


def _head(s4, s3, cnt, P1w, P1b, P2w, P2b, P3w, P3b):
    return pl.pallas_call(
        _head_body,
        out_shape=jax.ShapeDtypeStruct((NG, 1), jnp.float32),
    )(s4, s3, cnt, P1w, P1b.reshape(1, -1), P2w, P2b.reshape(1, -1),
      P3w, P3b.reshape(1, -1))


def kernel(x, edge_index, batch, W1, b1, g1, be1, W2, b2, g2, be2,
           W3, b3, g3, be3, W4, b4, P1w, P1b, P2w, P2b, P3w, P3b):
    src = edge_index[0]
    dst = edge_index[1]
    ones16 = jnp.ones((NN, 16), jnp.float32)

    deg16 = _prop_deg(ones16, src, dst)
    degcol = deg16[:, 0:1]

    b1r, g1r, be1r = b1.reshape(1, -1), g1.reshape(1, -1), be1.reshape(1, -1)
    b2r, g2r, be2r = b2.reshape(1, -1), g2.reshape(1, -1), be2.reshape(1, -1)
    b3r, g3r, be3r = b3.reshape(1, -1), g3.reshape(1, -1), be3.reshape(1, -1)
    b4r = b4.reshape(1, -1)

    hp1 = _matmul_scale(x, W1, degcol)
    P1 = _prop128(hp1, src, dst)
    x1 = _bn_relu(P1, degcol, b1r, g1r, be1r, _reduce_sums(P1, degcol, b1r))

    hp2 = _matmul_scale(x1, W2, degcol)
    P2 = _prop128(hp2, src, dst)
    x2 = _bn_relu(P2, degcol, b2r, g2r, be2r, _reduce_sums(P2, degcol, b2r),
                  res=x1)

    hp3 = _matmul_scale(x2, W3, degcol)
    P3 = _prop64(hp3, src, dst)
    x3 = _bn_relu(P3, degcol, b3r, g3r, be3r, _reduce_sums(P3, degcol, b3r))

    hp4 = _matmul_scale(x3, W4, degcol)
    P4 = _prop32(hp4, src, dst)
    x4 = _scale_relu(P4, degcol, b4r)

    s4, s3, cnt = _pool(x4, x3, batch.reshape(NN, 1))
    return _head(s4, s3, cnt, P1w, P1b, P2w, P2b, P3w, P3b)
